# Initial kernel scaffold; baseline (speedup 1.0000x reference)
import jax
import jax.numpy as jnp
from jax.experimental import pallas as pl


def kernel(x, edge_index, edge_label_index, W1, b1, Wmu, bmu, Wlv, blv, dW1, db1, dW2, db2):
    raise NotImplementedError("write your pallas kernel here")


# speedup vs baseline: 8.0140x; 8.0140x over previous
import functools

import jax
import jax.numpy as jnp
from jax import lax
from jax.experimental import pallas as pl
from jax.experimental.pallas import tpu as pltpu
from jax.experimental.pallas import tpu_sc as plsc

N = 10000
E = 320000
NP = 10240
NW = 32
NS = 16
CH = 128
NCH = 79
EC = NCH * CH
EPAD = NW * EC
STRIPE = NP // NS
BR = 2048
BRD = 4096

_mesh = plsc.VectorSubcoreMesh(core_axis_name="c", subcore_axis_name="s")
_sc_params = pltpu.CompilerParams(needs_layout_passes=False)
_sc_params_untiled = pltpu.CompilerParams(needs_layout_passes=False,
                                          use_tc_tiling_on_sc=False)


@functools.partial(
    pl.kernel,
    out_type=jax.ShapeDtypeStruct((NW, NP), jnp.float32),
    mesh=_mesh,
    compiler_params=_sc_params,
    scratch_types=[
        pltpu.VMEM((EC,), jnp.int32),
        pltpu.VMEM((NP,), jnp.float32),
    ],
)
def _sc_degree(dst_hbm, out_hbm, idx_v, hist_v):
    wid = lax.axis_index("c") * NS + lax.axis_index("s")
    pltpu.sync_copy(dst_hbm.at[wid], idx_v)
    zeros16 = jnp.zeros((16,), jnp.float32)
    ones16 = jnp.ones((16,), jnp.float32)

    def zbody(i, c):
        hist_v[pl.ds(i * 16, 16)] = zeros16
        return c

    lax.fori_loop(0, NP // 16, zbody, 0)

    def body(i, c):
        idx = idx_v[pl.ds(i * 16, 16)]
        plsc.addupdate_scatter(hist_v, [idx], ones16)
        return c

    lax.fori_loop(0, EC // 16, body, 0)
    pltpu.sync_copy(hist_v, out_hbm.at[wid])


@functools.partial(
    pl.kernel,
    out_type=jax.ShapeDtypeStruct((2 * NP, 128), jnp.float32),
    mesh=_mesh,
    scratch_types=[
        pltpu.VMEM((NCH, CH), jnp.int32),
        pltpu.VMEM((NCH, CH), jnp.int32),
        pltpu.VMEM((CH, 128), jnp.float32),
        pltpu.VMEM_SHARED((NP, 128), jnp.float32),
        pltpu.SemaphoreType.DMA,
    ],
)
def _sc_scatter(g_hbm, src_hbm, dst_hbm, zero_hbm, out_hbm,
                src_v, dst_v, rows_v, accum, sem):
    c = lax.axis_index("c")
    s = lax.axis_index("s")
    wid = c * NS + s
    pltpu.sync_copy(zero_hbm.at[pl.ds(s * STRIPE, STRIPE)],
                    accum.at[pl.ds(s * STRIPE, STRIPE)])
    plsc.subcore_barrier()
    pltpu.sync_copy(src_hbm.at[wid], src_v)
    pltpu.sync_copy(dst_hbm.at[wid], dst_v)

    def body(j, carry):
        pltpu.async_copy(g_hbm.at[src_v.at[j]], rows_v, sem).wait()
        pltpu.sync_copy(rows_v, accum.at[dst_v.at[j]], add=True)
        return carry

    lax.fori_loop(0, NCH, body, 0)
    plsc.subcore_barrier()
    pltpu.sync_copy(accum.at[pl.ds(s * STRIPE, STRIPE)],
                    out_hbm.at[pl.ds(c * NP + s * STRIPE, STRIPE)])


@functools.partial(
    pl.kernel,
    out_type=[jax.ShapeDtypeStruct((EPAD, 64), jnp.float32),
              jax.ShapeDtypeStruct((EPAD, 64), jnp.float32)],
    mesh=_mesh,
    compiler_params=_sc_params_untiled,
    scratch_types=[
        pltpu.VMEM((NCH, CH), jnp.int32),
        pltpu.VMEM((NCH, CH), jnp.int32),
        pltpu.VMEM((CH, 64), jnp.float32),
        pltpu.VMEM((CH, 64), jnp.float32),
        pltpu.SemaphoreType.DMA,
        pltpu.SemaphoreType.DMA,
    ],
)
def _sc_gather_pairs(z_hbm, e0_hbm, e1_hbm, outa_hbm, outb_hbm,
                     e0_v, e1_v, bufa, bufb, sema, semb):
    wid = lax.axis_index("c") * NS + lax.axis_index("s")
    base = wid * EC
    pltpu.sync_copy(e0_hbm.at[wid], e0_v)
    pltpu.sync_copy(e1_hbm.at[wid], e1_v)

    def body(j, carry):
        da = pltpu.async_copy(z_hbm.at[e0_v.at[j]], bufa, sema)
        db = pltpu.async_copy(z_hbm.at[e1_v.at[j]], bufb, semb)
        da.wait()
        db.wait()
        pltpu.sync_copy(bufa, outa_hbm.at[pl.ds(base + j * CH, CH)])
        pltpu.sync_copy(bufb, outb_hbm.at[pl.ds(base + j * CH, CH)])
        return carry

    lax.fori_loop(0, NCH, body, 0)


def _dinv_of(pt_block):
    deg = 1.0 + jnp.sum(pt_block, axis=1, keepdims=True)
    return lax.rsqrt(deg)


def _tc_encode1(pt, x, w1):
    def body(pt_ref, x_ref, w_ref, g1_ref):
        dinv = _dinv_of(pt_ref[...])
        h = lax.dot_general(x_ref[...], w_ref[...], (((1,), (1,)), ((), ())),
                            preferred_element_type=jnp.float32)
        g1_ref[...] = h * dinv

    return pl.pallas_call(
        body,
        grid=(NP // BR,),
        in_specs=[pl.BlockSpec((BR, NW), lambda i: (i, 0)),
                  pl.BlockSpec((BR, 128), lambda i: (i, 0)),
                  pl.BlockSpec((128, 128), lambda i: (0, 0))],
        out_specs=pl.BlockSpec((BR, 128), lambda i: (i, 0)),
        out_shape=jax.ShapeDtypeStruct((NP, 128), jnp.float32),
    )(pt, x, w1)


def _tc_encode2(pt, p0, p1, g1, b1, wcat):
    def body(pt_ref, p0_ref, p1_ref, g1_ref, b1_ref, w_ref, g2_ref):
        dinv = _dinv_of(pt_ref[...])
        h = jnp.maximum(dinv * (p0_ref[...] + p1_ref[...] + g1_ref[...])
                        + b1_ref[...], 0.0)
        g2 = lax.dot_general(h, w_ref[...], (((1,), (1,)), ((), ())),
                             preferred_element_type=jnp.float32)
        g2_ref[...] = g2 * dinv

    return pl.pallas_call(
        body,
        grid=(NP // BR,),
        in_specs=[pl.BlockSpec((BR, NW), lambda i: (i, 0)),
                  pl.BlockSpec((BR, 128), lambda i: (i, 0)),
                  pl.BlockSpec((BR, 128), lambda i: (i, 0)),
                  pl.BlockSpec((BR, 128), lambda i: (i, 0)),
                  pl.BlockSpec((1, 128), lambda i: (0, 0)),
                  pl.BlockSpec((128, 128), lambda i: (0, 0))],
        out_specs=pl.BlockSpec((BR, 128), lambda i: (i, 0)),
        out_shape=jax.ShapeDtypeStruct((NP, 128), jnp.float32),
    )(pt, p0, p1, g1, b1, wcat)


def _tc_reparam(pt, q0, q1, g2, bcat, eps):
    def body(pt_ref, q0_ref, q1_ref, g2_ref, b_ref, eps_ref,
             mu_ref, lv_ref, z_ref):
        dinv = _dinv_of(pt_ref[...])
        pre = dinv * (q0_ref[...] + q1_ref[...] + g2_ref[...]) + b_ref[...]
        mu = pre[:, :64]
        lv = pre[:, 64:]
        mu_ref[...] = mu
        lv_ref[...] = lv
        z_ref[...] = mu + eps_ref[...] * jnp.exp(0.5 * lv)

    o64 = pl.BlockSpec((BR, 64), lambda i: (i, 0))
    return pl.pallas_call(
        body,
        grid=(NP // BR,),
        in_specs=[pl.BlockSpec((BR, NW), lambda i: (i, 0)),
                  pl.BlockSpec((BR, 128), lambda i: (i, 0)),
                  pl.BlockSpec((BR, 128), lambda i: (i, 0)),
                  pl.BlockSpec((BR, 128), lambda i: (i, 0)),
                  pl.BlockSpec((1, 128), lambda i: (0, 0)),
                  pl.BlockSpec((BR, 64), lambda i: (i, 0))],
        out_specs=[o64, o64, o64],
        out_shape=[jax.ShapeDtypeStruct((NP, 64), jnp.float32)] * 3,
    )(pt, q0, q1, g2, bcat, eps)


def _tc_decode(fa, fb, a2, b2, db1, dw2, db2):
    def body(fa_ref, fb_ref, a_ref, b_ref, db1_ref, w2_ref, db2_ref, o_ref):
        hd = (lax.dot_general(fa_ref[...], a_ref[...], (((1,), (1,)), ((), ())),
                              preferred_element_type=jnp.float32)
              + lax.dot_general(fb_ref[...], b_ref[...], (((1,), (1,)), ((), ())),
                                preferred_element_type=jnp.float32)
              + db1_ref[...])
        hd = jnp.maximum(hd, 0.0)
        t = jnp.sum(hd * w2_ref[...], axis=1, keepdims=True)
        o_ref[...] = jax.nn.sigmoid(t + db2_ref[0, 0])

    return pl.pallas_call(
        body,
        grid=(EPAD // BRD,),
        in_specs=[pl.BlockSpec((BRD, 64), lambda i: (i, 0)),
                  pl.BlockSpec((BRD, 64), lambda i: (i, 0)),
                  pl.BlockSpec((128, 64), lambda i: (0, 0)),
                  pl.BlockSpec((128, 64), lambda i: (0, 0)),
                  pl.BlockSpec((1, 128), lambda i: (0, 0)),
                  pl.BlockSpec((1, 128), lambda i: (0, 0)),
                  pl.BlockSpec((1, 1), lambda i: (0, 0))],
        out_specs=pl.BlockSpec((BRD, 1), lambda i: (i, 0)),
        out_shape=jax.ShapeDtypeStruct((EPAD, 1), jnp.float32),
    )(fa, fb, a2, b2, db1, dw2, db2)


def kernel(x, edge_index, edge_label_index, W1, b1, Wmu, bmu, Wlv, blv,
           dW1, db1, dW2, db2):
    src = edge_index[0].astype(jnp.int32)
    dst = edge_index[1].astype(jnp.int32)
    e0 = edge_label_index[0].astype(jnp.int32)
    e1 = edge_label_index[1].astype(jnp.int32)

    pad = EPAD - E
    src_p = jnp.concatenate([src, jnp.zeros((pad,), jnp.int32)])
    dst_p = jnp.concatenate([dst, jnp.full((pad,), N, jnp.int32)])
    e0_p = jnp.concatenate([e0, jnp.zeros((pad,), jnp.int32)])
    e1_p = jnp.concatenate([e1, jnp.zeros((pad,), jnp.int32)])
    src3 = src_p.reshape(NW, NCH, CH)
    dst3 = dst_p.reshape(NW, NCH, CH)
    e0_3 = e0_p.reshape(NW, NCH, CH)
    e1_3 = e1_p.reshape(NW, NCH, CH)
    dst2 = dst_p.reshape(NW, EC)

    x_p = jnp.zeros((NP, 128), jnp.float32).at[:N].set(x)
    zero_rows = jnp.zeros((NP, 128), jnp.float32)
    wcat = jnp.concatenate([Wmu, Wlv], axis=0)
    bcat = jnp.concatenate([bmu, blv]).reshape(1, 128)
    eps = jax.random.normal(jax.random.key(42), (N, 64), dtype=jnp.float32)
    eps_p = jnp.zeros((NP, 64), jnp.float32).at[:N].set(eps)

    deg_parts = _sc_degree(dst2)
    pt = deg_parts.T

    g1 = _tc_encode1(pt, x_p, W1)
    parts1 = _sc_scatter(g1, src3, dst3, zero_rows)
    g2 = _tc_encode2(pt, parts1[:NP], parts1[NP:], g1,
                     b1.reshape(1, 128), wcat)
    parts2 = _sc_scatter(g2, src3, dst3, zero_rows)
    mu_p, lv_p, z_p = _tc_reparam(pt, parts2[:NP], parts2[NP:], g2,
                                  bcat, eps_p)

    fa, fb = _sc_gather_pairs(z_p, e0_3, e1_3)
    xc_p = _tc_decode(fa, fb, dW1[:, :64], dW1[:, 64:],
                      db1.reshape(1, 128), dW2, db2.reshape(1, 1))

    return (mu_p[:N], lv_p[:N], z_p[:N], xc_p[:E, 0])


# speedup vs baseline: 8.9954x; 1.1225x over previous
import functools

import jax
import jax.numpy as jnp
from jax import lax
from jax.experimental import pallas as pl
from jax.experimental.pallas import tpu as pltpu
from jax.experimental.pallas import tpu_sc as plsc

N = 10000
E = 320000
NP = 10240
NW = 32
NS = 16
CH = 128
NCH = 79
EC = NCH * CH
EPAD = NW * EC
STRIPE = NP // NS
BR = 2048
BRD = 4096

_mesh = plsc.VectorSubcoreMesh(core_axis_name="c", subcore_axis_name="s")
_sc_params = pltpu.CompilerParams(needs_layout_passes=False)
_sc_params_untiled = pltpu.CompilerParams(needs_layout_passes=False,
                                          use_tc_tiling_on_sc=False)


@functools.partial(
    pl.kernel,
    out_type=jax.ShapeDtypeStruct((NW, NP), jnp.float32),
    mesh=_mesh,
    compiler_params=_sc_params,
    scratch_types=[
        pltpu.VMEM((EC,), jnp.int32),
        pltpu.VMEM((NP,), jnp.float32),
    ],
)
def _sc_degree(dst_hbm, out_hbm, idx_v, hist_v):
    wid = lax.axis_index("c") * NS + lax.axis_index("s")
    pltpu.sync_copy(dst_hbm.at[wid], idx_v)
    zeros16 = jnp.zeros((16,), jnp.float32)
    ones16 = jnp.ones((16,), jnp.float32)

    def zbody(i, c):
        hist_v[pl.ds(i * 16, 16)] = zeros16
        return c

    lax.fori_loop(0, NP // 16, zbody, 0)

    def body(i, c):
        idx = idx_v[pl.ds(i * 16, 16)]
        plsc.addupdate_scatter(hist_v, [idx], ones16)
        return c

    lax.fori_loop(0, EC // 16, body, 0)
    pltpu.sync_copy(hist_v, out_hbm.at[wid])


DEPTH = 4
SDEPTH = 2
HCH = 40


@functools.partial(
    pl.kernel,
    out_type=jax.ShapeDtypeStruct((2 * NP, 128), jnp.float32),
    mesh=_mesh,
    scratch_types=[
        pltpu.VMEM((HCH, CH), jnp.int32),
        pltpu.VMEM((HCH, CH), jnp.int32),
        [pltpu.VMEM((CH, 128), jnp.float32) for _ in range(SDEPTH)],
        pltpu.VMEM_SHARED((NP, 128), jnp.float32),
        [pltpu.SemaphoreType.DMA for _ in range(SDEPTH)],
    ],
)
def _sc_scatter(g_hbm, src_hbm, dst_hbm, zero_hbm, out_hbm,
                src_v, dst_v, rows_v, accum, sem):
    c = lax.axis_index("c")
    s = lax.axis_index("s")
    wid = c * NS + s
    pltpu.sync_copy(zero_hbm.at[pl.ds(s * STRIPE, STRIPE)],
                    accum.at[pl.ds(s * STRIPE, STRIPE)])
    plsc.subcore_barrier()

    def phase(lo, n):
        pltpu.sync_copy(src_hbm.at[wid, pl.ds(lo, n)], src_v.at[pl.ds(0, n)])
        pltpu.sync_copy(dst_hbm.at[wid, pl.ds(lo, n)], dst_v.at[pl.ds(0, n)])
        for b in range(SDEPTH):
            pltpu.async_copy(g_hbm.at[src_v.at[b]], rows_v[b], sem[b])

        def outer(o, carry):
            j = o * SDEPTH
            for b in range(SDEPTH):
                tt = j + b

                @pl.when(tt < n)
                def _drain():
                    pltpu.make_async_copy(g_hbm.at[src_v.at[tt]], rows_v[b],
                                          sem[b]).wait()
                    pltpu.sync_copy(rows_v[b], accum.at[dst_v.at[tt]],
                                    add=True)

                nt = tt + SDEPTH

                @pl.when(nt < n)
                def _refill():
                    pltpu.async_copy(g_hbm.at[src_v.at[nt]], rows_v[b],
                                     sem[b])
            return carry

        lax.fori_loop(0, (n + SDEPTH - 1) // SDEPTH, outer, 0)

    phase(0, HCH)
    phase(HCH, NCH - HCH)
    plsc.subcore_barrier()
    pltpu.sync_copy(accum.at[pl.ds(s * STRIPE, STRIPE)],
                    out_hbm.at[pl.ds(c * NP + s * STRIPE, STRIPE)])


@functools.partial(
    pl.kernel,
    out_type=[jax.ShapeDtypeStruct((EPAD, 64), jnp.float32),
              jax.ShapeDtypeStruct((EPAD, 64), jnp.float32)],
    mesh=_mesh,
    compiler_params=_sc_params_untiled,
    scratch_types=[
        pltpu.VMEM((NCH, CH), jnp.int32),
        pltpu.VMEM((NCH, CH), jnp.int32),
        [pltpu.VMEM((CH, 64), jnp.float32) for _ in range(DEPTH)],
        [pltpu.VMEM((CH, 64), jnp.float32) for _ in range(DEPTH)],
        [pltpu.SemaphoreType.DMA for _ in range(DEPTH)],
        [pltpu.SemaphoreType.DMA for _ in range(DEPTH)],
    ],
)
def _sc_gather_pairs(z_hbm, e0_hbm, e1_hbm, outa_hbm, outb_hbm,
                     e0_v, e1_v, bufa, bufb, sema, semb):
    wid = lax.axis_index("c") * NS + lax.axis_index("s")
    base = wid * EC
    pltpu.sync_copy(e0_hbm.at[wid], e0_v)
    pltpu.sync_copy(e1_hbm.at[wid], e1_v)

    for b in range(DEPTH):
        pltpu.async_copy(z_hbm.at[e0_v.at[b]], bufa[b], sema[b])
        pltpu.async_copy(z_hbm.at[e1_v.at[b]], bufb[b], semb[b])

    def outer(o, carry):
        j = o * DEPTH
        for b in range(DEPTH):
            t = j + b

            @pl.when(t < NCH)
            def _drain():
                pltpu.make_async_copy(z_hbm.at[e0_v.at[t]], bufa[b],
                                      sema[b]).wait()
                pltpu.make_async_copy(z_hbm.at[e1_v.at[t]], bufb[b],
                                      semb[b]).wait()
                pltpu.sync_copy(bufa[b],
                                outa_hbm.at[pl.ds(base + t * CH, CH)])
                pltpu.sync_copy(bufb[b],
                                outb_hbm.at[pl.ds(base + t * CH, CH)])

            nt = t + DEPTH

            @pl.when(nt < NCH)
            def _refill():
                pltpu.async_copy(z_hbm.at[e0_v.at[nt]], bufa[b],
                                 sema[b])
                pltpu.async_copy(z_hbm.at[e1_v.at[nt]], bufb[b],
                                 semb[b])
        return carry

    lax.fori_loop(0, (NCH + DEPTH - 1) // DEPTH, outer, 0)


def _dinv_of(pt_block):
    deg = 1.0 + jnp.sum(pt_block, axis=1, keepdims=True)
    return lax.rsqrt(deg)


def _tc_encode1(pt, x, w1):
    def body(pt_ref, x_ref, w_ref, g1_ref):
        dinv = _dinv_of(pt_ref[...])
        h = lax.dot_general(x_ref[...], w_ref[...], (((1,), (1,)), ((), ())),
                            preferred_element_type=jnp.float32)
        g1_ref[...] = h * dinv

    return pl.pallas_call(
        body,
        grid=(NP // BR,),
        in_specs=[pl.BlockSpec((BR, NW), lambda i: (i, 0)),
                  pl.BlockSpec((BR, 128), lambda i: (i, 0)),
                  pl.BlockSpec((128, 128), lambda i: (0, 0))],
        out_specs=pl.BlockSpec((BR, 128), lambda i: (i, 0)),
        out_shape=jax.ShapeDtypeStruct((NP, 128), jnp.float32),
    )(pt, x, w1)


def _tc_encode2(pt, p0, p1, g1, b1, wcat):
    def body(pt_ref, p0_ref, p1_ref, g1_ref, b1_ref, w_ref, g2_ref):
        dinv = _dinv_of(pt_ref[...])
        h = jnp.maximum(dinv * (p0_ref[...] + p1_ref[...] + g1_ref[...])
                        + b1_ref[...], 0.0)
        g2 = lax.dot_general(h, w_ref[...], (((1,), (1,)), ((), ())),
                             preferred_element_type=jnp.float32)
        g2_ref[...] = g2 * dinv

    return pl.pallas_call(
        body,
        grid=(NP // BR,),
        in_specs=[pl.BlockSpec((BR, NW), lambda i: (i, 0)),
                  pl.BlockSpec((BR, 128), lambda i: (i, 0)),
                  pl.BlockSpec((BR, 128), lambda i: (i, 0)),
                  pl.BlockSpec((BR, 128), lambda i: (i, 0)),
                  pl.BlockSpec((1, 128), lambda i: (0, 0)),
                  pl.BlockSpec((128, 128), lambda i: (0, 0))],
        out_specs=pl.BlockSpec((BR, 128), lambda i: (i, 0)),
        out_shape=jax.ShapeDtypeStruct((NP, 128), jnp.float32),
    )(pt, p0, p1, g1, b1, wcat)


def _tc_reparam(pt, q0, q1, g2, bcat, eps):
    def body(pt_ref, q0_ref, q1_ref, g2_ref, b_ref, eps_ref,
             mu_ref, lv_ref, z_ref):
        dinv = _dinv_of(pt_ref[...])
        pre = dinv * (q0_ref[...] + q1_ref[...] + g2_ref[...]) + b_ref[...]
        mu = pre[:, :64]
        lv = pre[:, 64:]
        mu_ref[...] = mu
        lv_ref[...] = lv
        z_ref[...] = mu + eps_ref[...] * jnp.exp(0.5 * lv)

    o64 = pl.BlockSpec((BR, 64), lambda i: (i, 0))
    return pl.pallas_call(
        body,
        grid=(NP // BR,),
        in_specs=[pl.BlockSpec((BR, NW), lambda i: (i, 0)),
                  pl.BlockSpec((BR, 128), lambda i: (i, 0)),
                  pl.BlockSpec((BR, 128), lambda i: (i, 0)),
                  pl.BlockSpec((BR, 128), lambda i: (i, 0)),
                  pl.BlockSpec((1, 128), lambda i: (0, 0)),
                  pl.BlockSpec((BR, 64), lambda i: (i, 0))],
        out_specs=[o64, o64, o64],
        out_shape=[jax.ShapeDtypeStruct((NP, 64), jnp.float32)] * 3,
    )(pt, q0, q1, g2, bcat, eps)


def _tc_decode(fa, fb, a2, b2, db1, dw2, db2):
    def body(fa_ref, fb_ref, a_ref, b_ref, db1_ref, w2_ref, db2_ref, o_ref):
        hd = (lax.dot_general(fa_ref[...], a_ref[...], (((1,), (1,)), ((), ())),
                              preferred_element_type=jnp.float32)
              + lax.dot_general(fb_ref[...], b_ref[...], (((1,), (1,)), ((), ())),
                                preferred_element_type=jnp.float32)
              + db1_ref[...])
        hd = jnp.maximum(hd, 0.0)
        t = jnp.sum(hd * w2_ref[...], axis=1, keepdims=True)
        o_ref[...] = jax.nn.sigmoid(t + db2_ref[0, 0])

    return pl.pallas_call(
        body,
        grid=(EPAD // BRD,),
        in_specs=[pl.BlockSpec((BRD, 64), lambda i: (i, 0)),
                  pl.BlockSpec((BRD, 64), lambda i: (i, 0)),
                  pl.BlockSpec((128, 64), lambda i: (0, 0)),
                  pl.BlockSpec((128, 64), lambda i: (0, 0)),
                  pl.BlockSpec((1, 128), lambda i: (0, 0)),
                  pl.BlockSpec((1, 128), lambda i: (0, 0)),
                  pl.BlockSpec((1, 1), lambda i: (0, 0))],
        out_specs=pl.BlockSpec((BRD, 1), lambda i: (i, 0)),
        out_shape=jax.ShapeDtypeStruct((EPAD, 1), jnp.float32),
    )(fa, fb, a2, b2, db1, dw2, db2)


def kernel(x, edge_index, edge_label_index, W1, b1, Wmu, bmu, Wlv, blv,
           dW1, db1, dW2, db2):
    src = edge_index[0].astype(jnp.int32)
    dst = edge_index[1].astype(jnp.int32)
    e0 = edge_label_index[0].astype(jnp.int32)
    e1 = edge_label_index[1].astype(jnp.int32)

    pad = EPAD - E
    src_p = jnp.concatenate([src, jnp.zeros((pad,), jnp.int32)])
    dst_p = jnp.concatenate([dst, jnp.full((pad,), N, jnp.int32)])
    e0_p = jnp.concatenate([e0, jnp.zeros((pad,), jnp.int32)])
    e1_p = jnp.concatenate([e1, jnp.zeros((pad,), jnp.int32)])
    src3 = src_p.reshape(NW, NCH, CH)
    dst3 = dst_p.reshape(NW, NCH, CH)
    e0_3 = e0_p.reshape(NW, NCH, CH)
    e1_3 = e1_p.reshape(NW, NCH, CH)
    dst2 = dst_p.reshape(NW, EC)

    x_p = jnp.zeros((NP, 128), jnp.float32).at[:N].set(x)
    zero_rows = jnp.zeros((NP, 128), jnp.float32)
    wcat = jnp.concatenate([Wmu, Wlv], axis=0)
    bcat = jnp.concatenate([bmu, blv]).reshape(1, 128)
    eps = jax.random.normal(jax.random.key(42), (N, 64), dtype=jnp.float32)
    eps_p = jnp.zeros((NP, 64), jnp.float32).at[:N].set(eps)

    deg_parts = _sc_degree(dst2)
    pt = deg_parts.T

    g1 = _tc_encode1(pt, x_p, W1)
    parts1 = _sc_scatter(g1, src3, dst3, zero_rows)
    g2 = _tc_encode2(pt, parts1[:NP], parts1[NP:], g1,
                     b1.reshape(1, 128), wcat)
    parts2 = _sc_scatter(g2, src3, dst3, zero_rows)
    mu_p, lv_p, z_p = _tc_reparam(pt, parts2[:NP], parts2[NP:], g2,
                                  bcat, eps_p)

    fa, fb = _sc_gather_pairs(z_p, e0_3, e1_3)
    xc_p = _tc_decode(fa, fb, dW1[:, :64], dW1[:, 64:],
                      db1.reshape(1, 128), dW2, db2.reshape(1, 1))

    return (mu_p[:N], lv_p[:N], z_p[:N], xc_p[:E, 0])


# speedup vs baseline: 9.5309x; 1.0595x over previous
import functools

import jax
import jax.numpy as jnp
from jax import lax
from jax.experimental import pallas as pl
from jax.experimental.pallas import tpu as pltpu
from jax.experimental.pallas import tpu_sc as plsc

N = 10000
E = 320000
NP = 10240
NW = 32
NS = 16
CH = 128
NCH = 79
EC = NCH * CH
EPAD = NW * EC
F0CH = 118
F1CH = 2 * NCH - F0CH
P0CH = 79
P1CH = 2 * NCH - P0CH
EC0 = P0CH * CH
EC1 = P1CH * CH
STRIPE = NP // NS
BR = 2048
BRD = 4096

_mesh = plsc.VectorSubcoreMesh(core_axis_name="c", subcore_axis_name="s")
_sc_params = pltpu.CompilerParams(needs_layout_passes=False)
_sc_params_untiled = pltpu.CompilerParams(needs_layout_passes=False,
                                          use_tc_tiling_on_sc=False)


@functools.partial(
    pl.kernel,
    out_type=jax.ShapeDtypeStruct((NW, NP), jnp.float32),
    mesh=_mesh,
    compiler_params=_sc_params,
    scratch_types=[
        pltpu.VMEM((EC,), jnp.int32),
        pltpu.VMEM((NP,), jnp.float32),
    ],
)
def _sc_degree(dst_hbm, out_hbm, idx_v, hist_v):
    wid = lax.axis_index("c") * NS + lax.axis_index("s")
    pltpu.sync_copy(dst_hbm.at[wid], idx_v)
    zeros16 = jnp.zeros((16,), jnp.float32)
    ones16 = jnp.ones((16,), jnp.float32)

    def zbody(i, c):
        hist_v[pl.ds(i * 16, 16)] = zeros16
        return c

    lax.fori_loop(0, NP // 16, zbody, 0)

    def body(i, c):
        idx = idx_v[pl.ds(i * 16, 16)]
        plsc.addupdate_scatter(hist_v, [idx], ones16)
        return c

    lax.fori_loop(0, EC // 16, body, 0)
    pltpu.sync_copy(hist_v, out_hbm.at[wid])


DEPTH = 4
SDEPTH = 2
HCH = 40


@functools.partial(
    pl.kernel,
    out_type=jax.ShapeDtypeStruct((2 * NP, 128), jnp.float32),
    mesh=_mesh,
    scratch_types=[
        pltpu.VMEM((HCH, CH), jnp.int32),
        pltpu.VMEM((HCH, CH), jnp.int32),
        [pltpu.VMEM((CH, 128), jnp.float32) for _ in range(SDEPTH)],
        pltpu.VMEM_SHARED((NP, 128), jnp.float32),
        [pltpu.SemaphoreType.DMA for _ in range(SDEPTH)],
    ],
)
def _sc_scatter(g_hbm, srca_hbm, dsta_hbm, srcb_hbm, dstb_hbm, zero_hbm,
                out_hbm, src_v, dst_v, rows_v, accum, sem):
    c = lax.axis_index("c")
    s = lax.axis_index("s")
    pltpu.sync_copy(zero_hbm.at[pl.ds(s * STRIPE, STRIPE)],
                    accum.at[pl.ds(s * STRIPE, STRIPE)])
    plsc.subcore_barrier()

    def phase(src_hbm, dst_hbm, lo, n):
        pltpu.sync_copy(src_hbm.at[s, pl.ds(lo, n)], src_v.at[pl.ds(0, n)])
        pltpu.sync_copy(dst_hbm.at[s, pl.ds(lo, n)], dst_v.at[pl.ds(0, n)])
        for b in range(SDEPTH):
            pltpu.async_copy(g_hbm.at[src_v.at[b]], rows_v[b], sem[b])

        def outer(o, carry):
            j = o * SDEPTH
            for b in range(SDEPTH):
                tt = j + b

                @pl.when(tt < n)
                def _drain():
                    pltpu.make_async_copy(g_hbm.at[src_v.at[tt]], rows_v[b],
                                          sem[b]).wait()
                    pltpu.sync_copy(rows_v[b], accum.at[dst_v.at[tt]],
                                    add=True)

                nt = tt + SDEPTH

                @pl.when(nt < n)
                def _refill():
                    pltpu.async_copy(g_hbm.at[src_v.at[nt]], rows_v[b],
                                     sem[b])
            return carry

        lax.fori_loop(0, (n + SDEPTH - 1) // SDEPTH, outer, 0)

    @pl.when(c == 0)
    def _core0():
        lo = 0
        while lo < F0CH:
            n = min(HCH, F0CH - lo)
            phase(srca_hbm, dsta_hbm, lo, n)
            lo += n

    @pl.when(c == 1)
    def _core1():
        lo = 0
        while lo < F1CH:
            n = min(HCH, F1CH - lo)
            phase(srcb_hbm, dstb_hbm, lo, n)
            lo += n

    plsc.subcore_barrier()
    pltpu.sync_copy(accum.at[pl.ds(s * STRIPE, STRIPE)],
                    out_hbm.at[pl.ds(c * NP + s * STRIPE, STRIPE)])


@functools.partial(
    pl.kernel,
    out_type=[jax.ShapeDtypeStruct((EPAD, 64), jnp.bfloat16),
              jax.ShapeDtypeStruct((EPAD, 64), jnp.bfloat16)],
    mesh=_mesh,
    compiler_params=_sc_params_untiled,
    scratch_types=[
        pltpu.VMEM((P0CH, CH), jnp.int32),
        pltpu.VMEM((P0CH, CH), jnp.int32),
        [pltpu.VMEM((CH, 64), jnp.bfloat16) for _ in range(DEPTH)],
        [pltpu.VMEM((CH, 64), jnp.bfloat16) for _ in range(DEPTH)],
        [pltpu.SemaphoreType.DMA for _ in range(DEPTH)],
        [pltpu.SemaphoreType.DMA for _ in range(DEPTH)],
    ],
)
def _sc_gather_pairs(z_hbm, e0a_hbm, e1a_hbm, e0b_hbm, e1b_hbm,
                     outa_hbm, outb_hbm,
                     e0_v, e1_v, bufa, bufb, sema, semb):
    c = lax.axis_index("c")
    s = lax.axis_index("s")

    def run(e0_hbm, e1_hbm, base, n):
        pltpu.sync_copy(e0_hbm.at[s], e0_v.at[pl.ds(0, n)])
        pltpu.sync_copy(e1_hbm.at[s], e1_v.at[pl.ds(0, n)])

        for b in range(DEPTH):
            pltpu.async_copy(z_hbm.at[e0_v.at[b]], bufa[b], sema[b])
            pltpu.async_copy(z_hbm.at[e1_v.at[b]], bufb[b], semb[b])

        def outer(o, carry):
            j = o * DEPTH
            for b in range(DEPTH):
                t = j + b

                @pl.when(t < n)
                def _drain():
                    pltpu.make_async_copy(z_hbm.at[e0_v.at[t]], bufa[b],
                                          sema[b]).wait()
                    pltpu.make_async_copy(z_hbm.at[e1_v.at[t]], bufb[b],
                                          semb[b]).wait()
                    pltpu.sync_copy(bufa[b],
                                    outa_hbm.at[pl.ds(base + t * CH, CH)])
                    pltpu.sync_copy(bufb[b],
                                    outb_hbm.at[pl.ds(base + t * CH, CH)])

                nt = t + DEPTH

                @pl.when(nt < n)
                def _refill():
                    pltpu.async_copy(z_hbm.at[e0_v.at[nt]], bufa[b],
                                     sema[b])
                    pltpu.async_copy(z_hbm.at[e1_v.at[nt]], bufb[b],
                                     semb[b])
            return carry

        lax.fori_loop(0, (n + DEPTH - 1) // DEPTH, outer, 0)

    @pl.when(c == 0)
    def _core0():
        run(e0a_hbm, e1a_hbm, s * EC0, P0CH)

    @pl.when(c == 1)
    def _core1():
        run(e0b_hbm, e1b_hbm, NS * EC0 + s * EC1, P1CH)


def _dinv_of(pt_block):
    deg = 1.0 + jnp.sum(pt_block, axis=1, keepdims=True)
    return lax.rsqrt(deg)


def _tc_encode1(pt, x, w1):
    def body(pt_ref, x_ref, w_ref, g1_ref):
        dinv = _dinv_of(pt_ref[...])
        h = lax.dot_general(x_ref[...], w_ref[...], (((1,), (1,)), ((), ())),
                            preferred_element_type=jnp.float32)
        g1_ref[...] = h * dinv

    return pl.pallas_call(
        body,
        grid=(NP // BR,),
        in_specs=[pl.BlockSpec((BR, NW), lambda i: (i, 0)),
                  pl.BlockSpec((BR, 128), lambda i: (i, 0)),
                  pl.BlockSpec((128, 128), lambda i: (0, 0))],
        out_specs=pl.BlockSpec((BR, 128), lambda i: (i, 0)),
        out_shape=jax.ShapeDtypeStruct((NP, 128), jnp.float32),
    )(pt, x, w1)


def _tc_encode2(pt, p0, p1, g1, b1, wcat):
    def body(pt_ref, p0_ref, p1_ref, g1_ref, b1_ref, w_ref, g2_ref):
        dinv = _dinv_of(pt_ref[...])
        h = jnp.maximum(dinv * (p0_ref[...] + p1_ref[...] + g1_ref[...])
                        + b1_ref[...], 0.0)
        g2 = lax.dot_general(h, w_ref[...], (((1,), (1,)), ((), ())),
                             preferred_element_type=jnp.float32)
        g2_ref[...] = g2 * dinv

    return pl.pallas_call(
        body,
        grid=(NP // BR,),
        in_specs=[pl.BlockSpec((BR, NW), lambda i: (i, 0)),
                  pl.BlockSpec((BR, 128), lambda i: (i, 0)),
                  pl.BlockSpec((BR, 128), lambda i: (i, 0)),
                  pl.BlockSpec((BR, 128), lambda i: (i, 0)),
                  pl.BlockSpec((1, 128), lambda i: (0, 0)),
                  pl.BlockSpec((128, 128), lambda i: (0, 0))],
        out_specs=pl.BlockSpec((BR, 128), lambda i: (i, 0)),
        out_shape=jax.ShapeDtypeStruct((NP, 128), jnp.float32),
    )(pt, p0, p1, g1, b1, wcat)


def _tc_reparam(pt, q0, q1, g2, bcat, eps):
    def body(pt_ref, q0_ref, q1_ref, g2_ref, b_ref, eps_ref,
             mu_ref, lv_ref, z_ref, zb_ref):
        dinv = _dinv_of(pt_ref[...])
        pre = dinv * (q0_ref[...] + q1_ref[...] + g2_ref[...]) + b_ref[...]
        mu = pre[:, :64]
        lv = pre[:, 64:]
        mu_ref[...] = mu
        lv_ref[...] = lv
        z = mu + eps_ref[...] * jnp.exp(0.5 * lv)
        z_ref[...] = z
        zb_ref[...] = z.astype(jnp.bfloat16)

    o64 = pl.BlockSpec((BR, 64), lambda i: (i, 0))
    return pl.pallas_call(
        body,
        grid=(NP // BR,),
        in_specs=[pl.BlockSpec((BR, NW), lambda i: (i, 0)),
                  pl.BlockSpec((BR, 128), lambda i: (i, 0)),
                  pl.BlockSpec((BR, 128), lambda i: (i, 0)),
                  pl.BlockSpec((BR, 128), lambda i: (i, 0)),
                  pl.BlockSpec((1, 128), lambda i: (0, 0)),
                  pl.BlockSpec((BR, 64), lambda i: (i, 0))],
        out_specs=[o64, o64, o64, o64],
        out_shape=[jax.ShapeDtypeStruct((NP, 64), jnp.float32)] * 3
        + [jax.ShapeDtypeStruct((NP, 64), jnp.bfloat16)],
    )(pt, q0, q1, g2, bcat, eps)


def _tc_decode(fa, fb, a2, b2, db1, dw2, db2):
    def body(fa_ref, fb_ref, a_ref, b_ref, db1_ref, w2_ref, db2_ref, o_ref):
        hd = (lax.dot_general(fa_ref[...], a_ref[...], (((1,), (1,)), ((), ())),
                              preferred_element_type=jnp.float32)
              + lax.dot_general(fb_ref[...], b_ref[...], (((1,), (1,)), ((), ())),
                                preferred_element_type=jnp.float32)
              + db1_ref[...])
        hd = jnp.maximum(hd, 0.0)
        t = jnp.sum(hd * w2_ref[...], axis=1, keepdims=True)
        o_ref[...] = jax.nn.sigmoid(t + db2_ref[0, 0])

    return pl.pallas_call(
        body,
        grid=(EPAD // BRD,),
        in_specs=[pl.BlockSpec((BRD, 64), lambda i: (i, 0)),
                  pl.BlockSpec((BRD, 64), lambda i: (i, 0)),
                  pl.BlockSpec((128, 64), lambda i: (0, 0)),
                  pl.BlockSpec((128, 64), lambda i: (0, 0)),
                  pl.BlockSpec((1, 128), lambda i: (0, 0)),
                  pl.BlockSpec((1, 128), lambda i: (0, 0)),
                  pl.BlockSpec((1, 1), lambda i: (0, 0))],
        out_specs=pl.BlockSpec((BRD, 1), lambda i: (i, 0)),
        out_shape=jax.ShapeDtypeStruct((EPAD, 1), jnp.float32),
    )(fa, fb, a2, b2, db1, dw2, db2)


def kernel(x, edge_index, edge_label_index, W1, b1, Wmu, bmu, Wlv, blv,
           dW1, db1, dW2, db2):
    src = edge_index[0].astype(jnp.int32)
    dst = edge_index[1].astype(jnp.int32)
    e0 = edge_label_index[0].astype(jnp.int32)
    e1 = edge_label_index[1].astype(jnp.int32)

    pad = EPAD - E
    src_p = jnp.concatenate([src, jnp.zeros((pad,), jnp.int32)])
    dst_p = jnp.concatenate([dst, jnp.full((pad,), N, jnp.int32)])
    e0_p = jnp.concatenate([e0, jnp.zeros((pad,), jnp.int32)])
    e1_p = jnp.concatenate([e1, jnp.zeros((pad,), jnp.int32)])
    scut = NS * F0CH * CH
    srcA = src_p[:scut].reshape(NS, F0CH, CH)
    srcB = src_p[scut:].reshape(NS, F1CH, CH)
    dstA = dst_p[:scut].reshape(NS, F0CH, CH)
    dstB = dst_p[scut:].reshape(NS, F1CH, CH)
    cut = NS * EC0
    e0A = e0_p[:cut].reshape(NS, P0CH, CH)
    e0B = e0_p[cut:].reshape(NS, P1CH, CH)
    e1A = e1_p[:cut].reshape(NS, P0CH, CH)
    e1B = e1_p[cut:].reshape(NS, P1CH, CH)
    dst2 = dst_p.reshape(NW, EC)

    x_p = jnp.zeros((NP, 128), jnp.float32).at[:N].set(x)
    zero_rows = jnp.zeros((NP, 128), jnp.float32)
    wcat = jnp.concatenate([Wmu, Wlv], axis=0)
    bcat = jnp.concatenate([bmu, blv]).reshape(1, 128)
    eps = jax.random.normal(jax.random.key(42), (N, 64), dtype=jnp.float32)
    eps_p = jnp.zeros((NP, 64), jnp.float32).at[:N].set(eps)

    deg_parts = _sc_degree(dst2)
    pt = deg_parts.T

    g1 = _tc_encode1(pt, x_p, W1)
    parts1 = _sc_scatter(g1, srcA, dstA, srcB, dstB, zero_rows)
    g2 = _tc_encode2(pt, parts1[:NP], parts1[NP:], g1,
                     b1.reshape(1, 128), wcat)
    parts2 = _sc_scatter(g2, srcA, dstA, srcB, dstB, zero_rows)
    mu_p, lv_p, z_p, zb_p = _tc_reparam(pt, parts2[:NP], parts2[NP:], g2,
                                        bcat, eps_p)

    fa, fb = _sc_gather_pairs(zb_p, e0A, e1A, e0B, e1B)
    xc_p = _tc_decode(fa, fb,
                      dW1[:, :64].astype(jnp.bfloat16),
                      dW1[:, 64:].astype(jnp.bfloat16),
                      db1.reshape(1, 128), dW2, db2.reshape(1, 1))

    return (mu_p[:N], lv_p[:N], z_p[:N], xc_p[:E, 0])


# speedup vs baseline: 11.5023x; 1.2069x over previous
import functools

import jax
import jax.numpy as jnp
from jax import lax
from jax.experimental import pallas as pl
from jax.experimental.pallas import tpu as pltpu
from jax.experimental.pallas import tpu_sc as plsc

N = 10000
E = 320000
NP = 10240
NW = 32
NS = 16
CH = 128
NCH = 79
EC = NCH * CH
EPAD = NW * EC
F0CH = 118
F1CH = 2 * NCH - F0CH
P0CH = 100
P1CH = 2 * NCH - P0CH
EC0 = P0CH * CH
EC1 = P1CH * CH
STRIPE = NP // NS
BR = 2048
BRD = 4096

_mesh = plsc.VectorSubcoreMesh(core_axis_name="c", subcore_axis_name="s")
_sc_params = pltpu.CompilerParams(needs_layout_passes=False)
_sc_params_untiled = pltpu.CompilerParams(needs_layout_passes=False,
                                          use_tc_tiling_on_sc=False)


@functools.partial(
    pl.kernel,
    out_type=jax.ShapeDtypeStruct((NW, NP), jnp.float32),
    mesh=_mesh,
    compiler_params=_sc_params,
    scratch_types=[
        pltpu.VMEM((EC,), jnp.int32),
        pltpu.VMEM((NP,), jnp.float32),
    ],
)
def _sc_degree(dst_hbm, out_hbm, idx_v, hist_v):
    wid = lax.axis_index("c") * NS + lax.axis_index("s")
    pltpu.sync_copy(dst_hbm.at[wid], idx_v)
    zeros16 = jnp.zeros((16,), jnp.float32)
    ones16 = jnp.ones((16,), jnp.float32)

    def zbody(i, c):
        hist_v[pl.ds(i * 16, 16)] = zeros16
        return c

    lax.fori_loop(0, NP // 16, zbody, 0)

    def body(i, c):
        idx = idx_v[pl.ds(i * 16, 16)]
        plsc.addupdate_scatter(hist_v, [idx], ones16)
        return c

    lax.fori_loop(0, EC // 16, body, 0)
    pltpu.sync_copy(hist_v, out_hbm.at[wid])


DEPTH = 4
SDEPTH = 2
HCH = 40


@functools.partial(
    pl.kernel,
    out_type=jax.ShapeDtypeStruct((2 * NP, 128), jnp.float32),
    mesh=_mesh,
    scratch_types=[
        pltpu.VMEM((HCH, CH), jnp.int32),
        pltpu.VMEM((HCH, CH), jnp.int32),
        [pltpu.VMEM((CH, 128), jnp.float32) for _ in range(SDEPTH)],
        pltpu.VMEM_SHARED((NP, 128), jnp.float32),
        [pltpu.SemaphoreType.DMA for _ in range(SDEPTH)],
    ],
)
def _sc_scatter(g_hbm, srca_hbm, dsta_hbm, srcb_hbm, dstb_hbm, zero_hbm,
                out_hbm, src_v, dst_v, rows_v, accum, sem):
    c = lax.axis_index("c")
    s = lax.axis_index("s")
    pltpu.sync_copy(zero_hbm.at[pl.ds(s * STRIPE, STRIPE)],
                    accum.at[pl.ds(s * STRIPE, STRIPE)])
    plsc.subcore_barrier()

    def phase(src_hbm, dst_hbm, lo, n):
        pltpu.sync_copy(src_hbm.at[s, pl.ds(lo, n)], src_v.at[pl.ds(0, n)])
        pltpu.sync_copy(dst_hbm.at[s, pl.ds(lo, n)], dst_v.at[pl.ds(0, n)])
        for b in range(SDEPTH):
            pltpu.async_copy(g_hbm.at[src_v.at[b]], rows_v[b], sem[b])

        def outer(o, carry):
            j = o * SDEPTH
            for b in range(SDEPTH):
                tt = j + b

                @pl.when(tt < n)
                def _drain():
                    pltpu.make_async_copy(g_hbm.at[src_v.at[tt]], rows_v[b],
                                          sem[b]).wait()
                    pltpu.sync_copy(rows_v[b], accum.at[dst_v.at[tt]],
                                    add=True)

                nt = tt + SDEPTH

                @pl.when(nt < n)
                def _refill():
                    pltpu.async_copy(g_hbm.at[src_v.at[nt]], rows_v[b],
                                     sem[b])
            return carry

        lax.fori_loop(0, (n + SDEPTH - 1) // SDEPTH, outer, 0)

    @pl.when(c == 0)
    def _core0():
        lo = 0
        while lo < F0CH:
            n = min(HCH, F0CH - lo)
            phase(srca_hbm, dsta_hbm, lo, n)
            lo += n

    @pl.when(c == 1)
    def _core1():
        lo = 0
        while lo < F1CH:
            n = min(HCH, F1CH - lo)
            phase(srcb_hbm, dstb_hbm, lo, n)
            lo += n

    plsc.subcore_barrier()
    pltpu.sync_copy(accum.at[pl.ds(s * STRIPE, STRIPE)],
                    out_hbm.at[pl.ds(c * NP + s * STRIPE, STRIPE)])


@functools.partial(
    pl.kernel,
    out_type=jax.ShapeDtypeStruct((EPAD, 128), jnp.float32),
    mesh=_mesh,
    compiler_params=_sc_params_untiled,
    scratch_types=[
        pltpu.VMEM((P0CH, CH), jnp.int32),
        pltpu.VMEM((P0CH, CH), jnp.int32),
        [pltpu.VMEM((CH, 64), jnp.float32) for _ in range(DEPTH)],
        [pltpu.VMEM((CH, 64), jnp.float32) for _ in range(DEPTH)],
        [pltpu.SemaphoreType.DMA for _ in range(DEPTH)],
        [pltpu.SemaphoreType.DMA for _ in range(DEPTH)],
    ],
)
def _sc_gather_pairs(z_hbm, e0a_hbm, e1a_hbm, e0b_hbm, e1b_hbm,
                     out_hbm,
                     e0_v, e1_v, bufa, bufb, sema, semb):
    c = lax.axis_index("c")
    s = lax.axis_index("s")

    def run(e0_hbm, e1_hbm, base, n):
        pltpu.sync_copy(e0_hbm.at[s], e0_v.at[pl.ds(0, n)])
        pltpu.sync_copy(e1_hbm.at[s], e1_v.at[pl.ds(0, n)])

        for b in range(DEPTH):
            pltpu.async_copy(z_hbm.at[e0_v.at[b]], bufa[b], sema[b])
            pltpu.async_copy(z_hbm.at[e1_v.at[b]], bufb[b], semb[b])

        def outer(o, carry):
            j = o * DEPTH
            for b in range(DEPTH):
                t = j + b

                @pl.when(t < n)
                def _drain():
                    pltpu.make_async_copy(z_hbm.at[e0_v.at[t]], bufa[b],
                                          sema[b]).wait()
                    pltpu.make_async_copy(z_hbm.at[e1_v.at[t]], bufb[b],
                                          semb[b]).wait()
                    pltpu.sync_copy(
                        bufa[b],
                        out_hbm.at[pl.ds(base + t * CH, CH), pl.ds(0, 64)])
                    pltpu.sync_copy(
                        bufb[b],
                        out_hbm.at[pl.ds(base + t * CH, CH), pl.ds(64, 64)])

                nt = t + DEPTH

                @pl.when(nt < n)
                def _refill():
                    pltpu.async_copy(z_hbm.at[e0_v.at[nt]], bufa[b],
                                     sema[b])
                    pltpu.async_copy(z_hbm.at[e1_v.at[nt]], bufb[b],
                                     semb[b])
            return carry

        lax.fori_loop(0, (n + DEPTH - 1) // DEPTH, outer, 0)

    @pl.when(c == 0)
    def _core0():
        run(e0a_hbm, e1a_hbm, s * EC0, P0CH)

    @pl.when(c == 1)
    def _core1():
        run(e0b_hbm, e1b_hbm, NS * EC0 + s * EC1, P1CH)


def _dinv_of(pt_block):
    deg = 1.0 + jnp.sum(pt_block, axis=1, keepdims=True)
    return lax.rsqrt(deg)


def _tc_encode1(pt, x, w1):
    def body(pt_ref, x_ref, w_ref, g1_ref):
        dinv = _dinv_of(pt_ref[...])
        h = lax.dot_general(x_ref[...], w_ref[...], (((1,), (1,)), ((), ())),
                            preferred_element_type=jnp.float32)
        g1_ref[...] = h * dinv

    return pl.pallas_call(
        body,
        grid=(NP // BR,),
        in_specs=[pl.BlockSpec((BR, NW), lambda i: (i, 0)),
                  pl.BlockSpec((BR, 128), lambda i: (i, 0)),
                  pl.BlockSpec((128, 128), lambda i: (0, 0))],
        out_specs=pl.BlockSpec((BR, 128), lambda i: (i, 0)),
        out_shape=jax.ShapeDtypeStruct((NP, 128), jnp.float32),
    )(pt, x, w1)


def _tc_encode2(pt, p0, p1, g1, b1, wcat):
    def body(pt_ref, p0_ref, p1_ref, g1_ref, b1_ref, w_ref, g2_ref):
        dinv = _dinv_of(pt_ref[...])
        h = jnp.maximum(dinv * (p0_ref[...] + p1_ref[...] + g1_ref[...])
                        + b1_ref[...], 0.0)
        g2 = lax.dot_general(h, w_ref[...], (((1,), (1,)), ((), ())),
                             preferred_element_type=jnp.float32)
        g2_ref[...] = g2 * dinv

    return pl.pallas_call(
        body,
        grid=(NP // BR,),
        in_specs=[pl.BlockSpec((BR, NW), lambda i: (i, 0)),
                  pl.BlockSpec((BR, 128), lambda i: (i, 0)),
                  pl.BlockSpec((BR, 128), lambda i: (i, 0)),
                  pl.BlockSpec((BR, 128), lambda i: (i, 0)),
                  pl.BlockSpec((1, 128), lambda i: (0, 0)),
                  pl.BlockSpec((128, 128), lambda i: (0, 0))],
        out_specs=pl.BlockSpec((BR, 128), lambda i: (i, 0)),
        out_shape=jax.ShapeDtypeStruct((NP, 128), jnp.float32),
    )(pt, p0, p1, g1, b1, wcat)


def _tc_reparam(pt, q0, q1, g2, bcat, eps):
    def body(pt_ref, q0_ref, q1_ref, g2_ref, b_ref, eps_ref,
             mu_ref, lv_ref, z_ref):
        dinv = _dinv_of(pt_ref[...])
        pre = dinv * (q0_ref[...] + q1_ref[...] + g2_ref[...]) + b_ref[...]
        mu = pre[:, :64]
        lv = pre[:, 64:]
        mu_ref[...] = mu
        lv_ref[...] = lv
        z_ref[...] = mu + eps_ref[...] * jnp.exp(0.5 * lv)

    o64 = pl.BlockSpec((BR, 64), lambda i: (i, 0))
    return pl.pallas_call(
        body,
        grid=(NP // BR,),
        in_specs=[pl.BlockSpec((BR, NW), lambda i: (i, 0)),
                  pl.BlockSpec((BR, 128), lambda i: (i, 0)),
                  pl.BlockSpec((BR, 128), lambda i: (i, 0)),
                  pl.BlockSpec((BR, 128), lambda i: (i, 0)),
                  pl.BlockSpec((1, 128), lambda i: (0, 0)),
                  pl.BlockSpec((BR, 64), lambda i: (i, 0))],
        out_specs=[o64, o64, o64],
        out_shape=[jax.ShapeDtypeStruct((NP, 64), jnp.float32)] * 3,
    )(pt, q0, q1, g2, bcat, eps)


def _tc_decode(feat, w1d, db1, dw2, db2):
    def body(f_ref, w_ref, db1_ref, w2_ref, db2_ref, o_ref):
        hd = lax.dot_general(f_ref[...], w_ref[...], (((1,), (1,)), ((), ())),
                             preferred_element_type=jnp.float32) + db1_ref[...]
        hd = jnp.maximum(hd, 0.0)
        t = jnp.sum(hd * w2_ref[...], axis=1, keepdims=True)
        o_ref[...] = jax.nn.sigmoid(t + db2_ref[0, 0])

    return pl.pallas_call(
        body,
        grid=(EPAD // BRD,),
        in_specs=[pl.BlockSpec((BRD, 128), lambda i: (i, 0)),
                  pl.BlockSpec((128, 128), lambda i: (0, 0)),
                  pl.BlockSpec((1, 128), lambda i: (0, 0)),
                  pl.BlockSpec((1, 128), lambda i: (0, 0)),
                  pl.BlockSpec((1, 1), lambda i: (0, 0))],
        out_specs=pl.BlockSpec((BRD, 1), lambda i: (i, 0)),
        out_shape=jax.ShapeDtypeStruct((EPAD, 1), jnp.float32),
    )(feat, w1d, db1, dw2, db2)


def kernel(x, edge_index, edge_label_index, W1, b1, Wmu, bmu, Wlv, blv,
           dW1, db1, dW2, db2):
    src = edge_index[0].astype(jnp.int32)
    dst = edge_index[1].astype(jnp.int32)
    e0 = edge_label_index[0].astype(jnp.int32)
    e1 = edge_label_index[1].astype(jnp.int32)

    pad = EPAD - E
    src_p = jnp.concatenate([src, jnp.zeros((pad,), jnp.int32)])
    dst_p = jnp.concatenate([dst, jnp.full((pad,), N, jnp.int32)])
    e0_p = jnp.concatenate([e0, jnp.zeros((pad,), jnp.int32)])
    e1_p = jnp.concatenate([e1, jnp.zeros((pad,), jnp.int32)])
    scut = NS * F0CH * CH
    srcA = src_p[:scut].reshape(NS, F0CH, CH)
    srcB = src_p[scut:].reshape(NS, F1CH, CH)
    dstA = dst_p[:scut].reshape(NS, F0CH, CH)
    dstB = dst_p[scut:].reshape(NS, F1CH, CH)
    cut = NS * EC0
    e0A = e0_p[:cut].reshape(NS, P0CH, CH)
    e0B = e0_p[cut:].reshape(NS, P1CH, CH)
    e1A = e1_p[:cut].reshape(NS, P0CH, CH)
    e1B = e1_p[cut:].reshape(NS, P1CH, CH)
    dst2 = dst_p.reshape(NW, EC)

    x_p = jnp.zeros((NP, 128), jnp.float32).at[:N].set(x)
    zero_rows = jnp.zeros((NP, 128), jnp.float32)
    wcat = jnp.concatenate([Wmu, Wlv], axis=0)
    bcat = jnp.concatenate([bmu, blv]).reshape(1, 128)
    eps = jax.random.normal(jax.random.key(42), (N, 64), dtype=jnp.float32)
    eps_p = jnp.zeros((NP, 64), jnp.float32).at[:N].set(eps)

    deg_parts = _sc_degree(dst2)
    pt = deg_parts.T

    g1 = _tc_encode1(pt, x_p, W1)
    parts1 = _sc_scatter(g1, srcA, dstA, srcB, dstB, zero_rows)
    g2 = _tc_encode2(pt, parts1[:NP], parts1[NP:], g1,
                     b1.reshape(1, 128), wcat)
    parts2 = _sc_scatter(g2, srcA, dstA, srcB, dstB, zero_rows)
    mu_p, lv_p, z_p = _tc_reparam(pt, parts2[:NP], parts2[NP:], g2,
                                  bcat, eps_p)

    feat = _sc_gather_pairs(z_p, e0A, e1A, e0B, e1B)
    xc_p = _tc_decode(feat, dW1, db1.reshape(1, 128), dW2,
                      db2.reshape(1, 1))

    return (mu_p[:N], lv_p[:N], z_p[:N], xc_p[:E, 0])
